# baseline (speedup 1.0000x reference)
import functools

import jax
import jax.numpy as jnp
from jax import lax
from jax.experimental import pallas as pl
from jax.experimental.pallas import tpu as pltpu
from jax.experimental.pallas import tpu_sc as plsc

_N = 10000
_E = 160000
_D = 128
_G3 = 384

_F32 = jnp.float32
_PREC = lax.Precision.HIGHEST


def _dott(a, b):
    return lax.dot_general(a, b, (((1,), (1,)), ((), ())),
                           precision=_PREC, preferred_element_type=_F32)


def _dot(a, b):
    return lax.dot_general(a, b, (((1,), (0,)), ((), ())),
                           precision=_PREC, preferred_element_type=_F32)


def _dot_rec(a, b_bf16):
    return lax.dot_general(a.astype(jnp.bfloat16), b_bf16,
                           (((1,), (0,)), ((), ())),
                           precision=lax.Precision.DEFAULT,
                           preferred_element_type=_F32)


def _gates(gi, gh, hprev):
    r = jax.nn.sigmoid(gi[:, :_D] + gh[:, :_D])
    z = jax.nn.sigmoid(gi[:, _D:2 * _D] + gh[:, _D:2 * _D])
    n = jnp.tanh(gi[:, 2 * _D:] + r * gh[:, 2 * _D:])
    return (1.0 - z) * n + z * hprev


def _gather_body(ids_ref, glove_ref, out_ref):
    del ids_ref
    out_ref[...] = glove_ref[...]


def _gather_rows(glove, ids):
    L = ids.shape[0]
    glove3 = glove.reshape(glove.shape[0], 1, _D)
    grid_spec = pltpu.PrefetchScalarGridSpec(
        num_scalar_prefetch=1,
        grid=(L,),
        in_specs=[pl.BlockSpec((1, 1, _D), lambda i, ids: (ids[i], 0, 0))],
        out_specs=pl.BlockSpec((1, 1, _D), lambda i, ids: (i, 0, 0)),
    )
    out = pl.pallas_call(
        _gather_body,
        grid_spec=grid_spec,
        out_shape=jax.ShapeDtypeStruct((L, 1, _D), _F32),
    )(ids, glove3)
    return out.reshape(L, _D)


def _title_body(emb_ref,
                w0f_ref, u0f_ref, bi0f_ref, bh0f_ref,
                w0b_ref, u0b_ref, bi0b_ref, bh0b_ref,
                w1f_ref, u1f_ref, bi1f_ref, bh1f_ref,
                w1b_ref, u1b_ref, bi1b_ref, bh1b_ref,
                title_ref,
                ysf_ref, ysb_ref, g1f_ref, g1b_ref, g0f_ref, g0b_ref):
    L = emb_ref.shape[0]
    emb = emb_ref[...]
    g0f_ref[...] = _dott(emb, w0f_ref[...]) + bi0f_ref[...]
    g0b_ref[...] = _dott(emb, w0b_ref[...]) + bi0b_ref[...]
    u0f = u0f_ref[...]
    u0b = u0b_ref[...]
    bh0f = bh0f_ref[...]
    bh0b = bh0b_ref[...]

    def loop0(k, carry):
        hf, hb = carry
        t = L - 1 - k
        hf = _gates(g0f_ref[pl.ds(k, 1), :], _dot(hf, u0f) + bh0f, hf)
        hb = _gates(g0b_ref[pl.ds(t, 1), :], _dot(hb, u0b) + bh0b, hb)
        ysf_ref[pl.ds(k, 1), :] = hf
        ysb_ref[pl.ds(t, 1), :] = hb
        return hf, hb

    z = jnp.zeros((1, _D), _F32)
    h0f, h0b = lax.fori_loop(0, L, loop0, (z, z))

    l1 = jnp.concatenate([ysf_ref[...], ysb_ref[...]], axis=1)
    g1f_ref[...] = _dott(l1, w1f_ref[...]) + bi1f_ref[...]
    g1b_ref[...] = _dott(l1, w1b_ref[...]) + bi1b_ref[...]
    u1f = u1f_ref[...]
    u1b = u1b_ref[...]
    bh1f = bh1f_ref[...]
    bh1b = bh1b_ref[...]

    def loop1(k, carry):
        hf, hb = carry
        t = L - 1 - k
        hf = _gates(g1f_ref[pl.ds(k, 1), :], _dot(hf, u1f) + bh1f, hf)
        hb = _gates(g1b_ref[pl.ds(t, 1), :], _dot(hb, u1b) + bh1b, hb)
        return hf, hb

    h1f, h1b = lax.fori_loop(0, L, loop1, (z, z))
    title_ref[...] = 0.25 * (h0f + h0b + h1f + h1b)


def _title_encode(emb, enc_params):
    L = emb.shape[0]
    flat = []
    for (wih, whh, bih, bhh) in enc_params:
        flat += [wih, whh.T, bih.reshape(1, -1), bhh.reshape(1, -1)]
    full = lambda s: pl.BlockSpec(s, lambda: (0,) * len(s))
    in_specs = [full((L, _D))]
    for (wih, whh, bih, bhh) in enc_params:
        in_specs += [full(wih.shape), full(whh.T.shape),
                     full((1, _G3)), full((1, _G3))]
    return pl.pallas_call(
        _title_body,
        grid=(),
        in_specs=in_specs,
        out_specs=full((1, _D)),
        out_shape=jax.ShapeDtypeStruct((1, _D), _F32),
        scratch_shapes=[
            pltpu.VMEM((L, _D), _F32),
            pltpu.VMEM((L, _D), _F32),
            pltpu.VMEM((L, _G3), _F32),
            pltpu.VMEM((L, _G3), _F32),
            pltpu.VMEM((L, _G3), _F32),
            pltpu.VMEM((L, _G3), _F32),
        ],
    )(emb, *flat)


_CHUNK = 1000
_NCHUNK = _N // _CHUNK


def _nodes_body(h_ref, title_ref, w0t_ref, w0h_ref, w1t_ref, bi0_ref,
                u0_ref, u1_ref, bh0_ref, bi1_ref, bh1_ref,
                f_ref, hx_ref,
                gi0_ref, gi1_ref, ys0_ref, h1_ref, h2_ref, tp_ref):
    c = pl.program_id(0)

    @pl.when(c == 0)
    def _init():
        h1_ref[...] = jnp.zeros((1, _D), _F32)
        h2_ref[...] = jnp.zeros((1, _D), _F32)
        tp_ref[...] = _dott(title_ref[...], w0t_ref[...]) + bi0_ref[...]

    gi0_ref[...] = _dott(h_ref[...], w0h_ref[...]) + tp_ref[...]

    u0 = u0_ref[...]
    u1 = u1_ref[...]
    bh0 = bh0_ref[...]
    bh1 = bh1_ref[...]

    h1p = h1_ref[...]
    h2p = h2_ref[...]

    def step(t, carry):
        h1c, h2c = carry
        gh1 = _dot_rec(h1c, u0) + bh0
        gh2 = _dot_rec(h2c, u1) + bh1
        h1n = _gates(gi0_ref[pl.ds(t, 1), :], gh1, h1c)
        h2n = _gates(gi1_ref[pl.ds(t, 1), :], gh2, h2c)
        ys0_ref[pl.ds(t, 1), :] = h1n
        f_ref[pl.ds(t, 1), :] = h2n
        return h1n, h2n

    h1c, h2c = lax.fori_loop(0, _CHUNK, step, (h1p, h2p), unroll=8)

    gi1_ref[...] = _dott(ys0_ref[...], w1t_ref[...]) + bi1_ref[...]

    h1_ref[...] = h1c
    h2_ref[...] = jnp.where(c == 0, jnp.zeros((1, _D), _F32), h2c)

    @pl.when(c == _NCHUNK)
    def _fin():
        hx_ref[...] = jnp.concatenate([h1p, h2c], axis=0)


def _nodes_scan(h, title, rnn_params):
    (wih0, whh0, bih0, bhh0), (wih1, whh1, bih1, bhh1) = rnn_params
    w0t = wih0[:, :_D]
    w0h = wih0[:, _D:]
    u0 = whh0.T.astype(jnp.bfloat16)
    u1 = whh1.T.astype(jnp.bfloat16)

    full = lambda s: pl.BlockSpec(s, lambda c: (0,) * len(s))
    f, hx = pl.pallas_call(
        _nodes_body,
        grid=(_NCHUNK + 1,),
        in_specs=[
            pl.BlockSpec((_CHUNK, _D),
                         lambda c: (jnp.minimum(c, _NCHUNK - 1), 0)),
            full((1, _D)),
            full((_G3, _D)),
            full((_G3, _D)),
            full((_G3, _D)),
            full((1, _G3)),
            full((_D, _G3)),
            full((_D, _G3)),
            full((1, _G3)),
            full((1, _G3)),
            full((1, _G3)),
        ],
        out_specs=[
            pl.BlockSpec((_CHUNK, _D),
                         lambda c: (jnp.maximum(c - 1, 0), 0)),
            full((2, _D)),
        ],
        out_shape=[
            jax.ShapeDtypeStruct((_N, _D), _F32),
            jax.ShapeDtypeStruct((2, _D), _F32),
        ],
        scratch_shapes=[
            pltpu.VMEM((_CHUNK, _G3), _F32),
            pltpu.VMEM((_CHUNK, _G3), _F32),
            pltpu.VMEM((_CHUNK, _D), _F32),
            pltpu.VMEM((1, _D), _F32),
            pltpu.VMEM((1, _D), _F32),
            pltpu.VMEM((1, _G3), _F32),
        ],
        compiler_params=pltpu.CompilerParams(
            dimension_semantics=("arbitrary",),
        ),
    )(h, title, w0t, w0h, wih1, bih0.reshape(1, -1), u0, u1,
      bhh0.reshape(1, -1), bih1.reshape(1, -1), bhh1.reshape(1, -1))
    return f, hx


_NW = 16
_EW = _E // _NW
_NPAD = 10240
_SL = _NPAD // _NW
_VL = 16


def _edge_body(src_hbm, dst_hbm, sid_hbm, out_hbm,
               src_v, dst_v, cnt_v, tmp_v, acc_v, invs_v, invf_v, pred_v,
               sid_v, part_sh, inv_sh):
    cid = lax.axis_index("c")

    @pl.when(cid == 0)
    def _run():
        w = lax.axis_index("s")
        base = w * _EW
        pltpu.sync_copy(sid_hbm, sid_v)
        pltpu.sync_copy(src_hbm.at[pl.ds(base, _EW)], src_v)
        pltpu.sync_copy(dst_hbm.at[pl.ds(base, _EW)], dst_v)
        s0 = sid_v[...]

        zi = jnp.zeros((_VL,), jnp.int32)

        def zbody(i, carry):
            cnt_v[pl.ds(i * _VL, _VL)] = zi
            return carry

        lax.fori_loop(0, _NPAD // _VL, zbody, 0)

        ones = jnp.ones((_VL,), jnp.int32)

        def hbody(i, carry):
            s = src_v[pl.ds(i * _VL, _VL)]
            d = dst_v[pl.ds(i * _VL, _VL)]
            plsc.addupdate_scatter(cnt_v, [d], ones, mask=s == s0)
            return carry

        lax.fori_loop(0, _EW // _VL, hbody, 0)

        pltpu.sync_copy(cnt_v, part_sh.at[w])
        plsc.subcore_barrier()

        colbase = w * _SL
        pltpu.sync_copy(part_sh.at[0, pl.ds(colbase, _SL)], acc_v)

        def rbody(j, carry):
            pltpu.sync_copy(part_sh.at[j, pl.ds(colbase, _SL)], tmp_v)

            def abody(i, c2):
                sl = pl.ds(i * _VL, _VL)
                acc_v[sl] = acc_v[sl] + tmp_v[sl]
                return c2

            lax.fori_loop(0, _SL // _VL, abody, 0)
            return carry

        lax.fori_loop(1, _NW, rbody, 0)

        def ibody(i, carry):
            sl = pl.ds(i * _VL, _VL)
            invs_v[sl] = 1.0 / acc_v[sl].astype(_F32)
            return carry

        lax.fori_loop(0, _SL // _VL, ibody, 0)

        pltpu.sync_copy(invs_v, inv_sh.at[pl.ds(colbase, _SL)])
        plsc.subcore_barrier()
        pltpu.sync_copy(inv_sh, invf_v)

        zf = jnp.zeros((_VL,), _F32)

        def pbody(i, carry):
            sl = pl.ds(i * _VL, _VL)
            s = src_v[sl]
            d = dst_v[sl]
            vals = plsc.load_gather(invf_v, [d])
            pred_v[sl] = jnp.where(s == s0, vals, zf)
            return carry

        lax.fori_loop(0, _EW // _VL, pbody, 0)
        pltpu.sync_copy(pred_v, out_hbm.at[pl.ds(base, _EW)])


def _edge_softmax(src, dst, sid16):
    mesh = plsc.VectorSubcoreMesh(core_axis_name="c", subcore_axis_name="s")
    run = pl.kernel(
        _edge_body,
        out_type=jax.ShapeDtypeStruct((_E,), _F32),
        mesh=mesh,
        scratch_types=[
            pltpu.VMEM((_EW,), jnp.int32),
            pltpu.VMEM((_EW,), jnp.int32),
            pltpu.VMEM((_NPAD,), jnp.int32),
            pltpu.VMEM((_SL,), jnp.int32),
            pltpu.VMEM((_SL,), jnp.int32),
            pltpu.VMEM((_SL,), _F32),
            pltpu.VMEM((_NPAD,), _F32),
            pltpu.VMEM((_EW,), _F32),
            pltpu.VMEM((_VL,), jnp.int32),
            pltpu.VMEM_SHARED((_NW, _NPAD), jnp.int32),
            pltpu.VMEM_SHARED((_NPAD,), _F32),
        ],
        compiler_params=pltpu.CompilerParams(needs_layout_passes=False),
    )
    return run(src, dst, sid16)


def kernel(h, edge_index, title_src, src_node_id, glove, enc_params, rnn_params):
    emb = _gather_rows(glove, title_src)
    title = _title_encode(emb, enc_params)
    f, hx = _nodes_scan(h, title, rnn_params)
    sid16 = jnp.full((_VL,), src_node_id, jnp.int32)
    pred = _edge_softmax(edge_index[0], edge_index[1], sid16)
    return pred, hx, f


# speedup vs baseline: 1.1989x; 1.1989x over previous
import functools

import jax
import jax.numpy as jnp
from jax import lax
from jax.experimental import pallas as pl
from jax.experimental.pallas import tpu as pltpu
from jax.experimental.pallas import tpu_sc as plsc

_N = 10000
_E = 160000
_D = 128
_G3 = 384

_F32 = jnp.float32
_PREC = lax.Precision.HIGHEST


def _dott(a, b):
    return lax.dot_general(a, b, (((1,), (1,)), ((), ())),
                           precision=_PREC, preferred_element_type=_F32)


def _dot(a, b):
    return lax.dot_general(a, b, (((1,), (0,)), ((), ())),
                           precision=_PREC, preferred_element_type=_F32)


def _dot_bf(a, b):
    return lax.dot_general(a, b, (((1,), (0,)), ((), ())),
                           precision=lax.Precision.DEFAULT,
                           preferred_element_type=_F32)


def _gates(gi, gh, hprev):
    r = jax.nn.sigmoid(gi[:, :_D] + gh[:, :_D])
    z = jax.nn.sigmoid(gi[:, _D:2 * _D] + gh[:, _D:2 * _D])
    n = jnp.tanh(gi[:, 2 * _D:] + r * gh[:, 2 * _D:])
    return (1.0 - z) * n + z * hprev


def _gather_body(ids_ref, glove_ref, out_ref):
    del ids_ref
    out_ref[...] = glove_ref[...]


def _gather_rows(glove, ids):
    L = ids.shape[0]
    glove3 = glove.reshape(glove.shape[0], 1, _D)
    grid_spec = pltpu.PrefetchScalarGridSpec(
        num_scalar_prefetch=1,
        grid=(L,),
        in_specs=[pl.BlockSpec((1, 1, _D), lambda i, ids: (ids[i], 0, 0))],
        out_specs=pl.BlockSpec((1, 1, _D), lambda i, ids: (i, 0, 0)),
    )
    out = pl.pallas_call(
        _gather_body,
        grid_spec=grid_spec,
        out_shape=jax.ShapeDtypeStruct((L, 1, _D), _F32),
    )(ids, glove3)
    return out.reshape(L, _D)


def _title_body(emb_ref,
                w0f_ref, u0f_ref, bi0f_ref, bh0f_ref,
                w0b_ref, u0b_ref, bi0b_ref, bh0b_ref,
                w1f_ref, u1f_ref, bi1f_ref, bh1f_ref,
                w1b_ref, u1b_ref, bi1b_ref, bh1b_ref,
                title_ref,
                ysf_ref, ysb_ref, g1f_ref, g1b_ref, g0f_ref, g0b_ref):
    L = emb_ref.shape[0]
    emb = emb_ref[...]
    g0f_ref[...] = _dott(emb, w0f_ref[...]) + bi0f_ref[...]
    g0b_ref[...] = _dott(emb, w0b_ref[...]) + bi0b_ref[...]
    u0f = u0f_ref[...]
    u0b = u0b_ref[...]
    bh0f = bh0f_ref[...]
    bh0b = bh0b_ref[...]

    def loop0(k, carry):
        hf, hb = carry
        t = L - 1 - k
        hf = _gates(g0f_ref[pl.ds(k, 1), :], _dot(hf, u0f) + bh0f, hf)
        hb = _gates(g0b_ref[pl.ds(t, 1), :], _dot(hb, u0b) + bh0b, hb)
        ysf_ref[pl.ds(k, 1), :] = hf
        ysb_ref[pl.ds(t, 1), :] = hb
        return hf, hb

    z = jnp.zeros((1, _D), _F32)
    h0f, h0b = lax.fori_loop(0, L, loop0, (z, z))

    l1 = jnp.concatenate([ysf_ref[...], ysb_ref[...]], axis=1)
    g1f_ref[...] = _dott(l1, w1f_ref[...]) + bi1f_ref[...]
    g1b_ref[...] = _dott(l1, w1b_ref[...]) + bi1b_ref[...]
    u1f = u1f_ref[...]
    u1b = u1b_ref[...]
    bh1f = bh1f_ref[...]
    bh1b = bh1b_ref[...]

    def loop1(k, carry):
        hf, hb = carry
        t = L - 1 - k
        hf = _gates(g1f_ref[pl.ds(k, 1), :], _dot(hf, u1f) + bh1f, hf)
        hb = _gates(g1b_ref[pl.ds(t, 1), :], _dot(hb, u1b) + bh1b, hb)
        return hf, hb

    h1f, h1b = lax.fori_loop(0, L, loop1, (z, z))
    title_ref[...] = 0.25 * (h0f + h0b + h1f + h1b)


def _title_encode(emb, enc_params):
    L = emb.shape[0]
    flat = []
    for (wih, whh, bih, bhh) in enc_params:
        flat += [wih, whh.T, bih.reshape(1, -1), bhh.reshape(1, -1)]
    full = lambda s: pl.BlockSpec(s, lambda: (0,) * len(s))
    in_specs = [full((L, _D))]
    for (wih, whh, bih, bhh) in enc_params:
        in_specs += [full(wih.shape), full(whh.T.shape),
                     full((1, _G3)), full((1, _G3))]
    return pl.pallas_call(
        _title_body,
        grid=(),
        in_specs=in_specs,
        out_specs=full((1, _D)),
        out_shape=jax.ShapeDtypeStruct((1, _D), _F32),
        scratch_shapes=[
            pltpu.VMEM((L, _D), _F32),
            pltpu.VMEM((L, _D), _F32),
            pltpu.VMEM((L, _G3), _F32),
            pltpu.VMEM((L, _G3), _F32),
            pltpu.VMEM((L, _G3), _F32),
            pltpu.VMEM((L, _G3), _F32),
        ],
    )(emb, *flat)


_CHUNK = 1000
_NCHUNK = _N // _CHUNK


def _gates3(gi, gr, gz, gn, hprev):
    r = 0.5 + 0.5 * jnp.tanh(gi[:, :_D] + gr)
    z = 0.5 + 0.5 * jnp.tanh(gi[:, _D:2 * _D] + gz)
    n = jnp.tanh(gi[:, 2 * _D:] + r * gn)
    return n + z * (hprev - n)


def _vpu_mv(hc, u):
    p = hc.reshape(_D, 1) * u
    p = p[0:64] + p[64:128]
    p = p[0:32] + p[32:64]
    p = p[0:16] + p[16:32]
    p = p[0:8] + p[8:16]
    return jnp.sum(p, axis=0, keepdims=True)


def _nodes_body(h_ref, title_ref, w0t_ref, w0h_ref, w1t_ref, bi0_ref,
                u0_ref, u1_ref, bh0n_ref, bi1_ref, bh1n_ref,
                f_ref, hx_ref,
                gi0_ref, gi1_ref, ys0_ref, h1_ref, h2_ref, tp_ref):
    c = pl.program_id(0)

    @pl.when(c == 0)
    def _init():
        h1_ref[...] = jnp.zeros((1, _D), _F32)
        h2_ref[...] = jnp.zeros((1, _D), _F32)
        tp_ref[...] = _dott(title_ref[...], w0t_ref[...]) + bi0_ref[...]

    gi0_ref[...] = _dott(h_ref[...], w0h_ref[...]) + tp_ref[...]

    u0 = u0_ref[...]
    u1 = u1_ref[...]
    bh0n = bh0n_ref[...]
    bh1n = bh1n_ref[...]

    h1p = h1_ref[...]
    h2p = h2_ref[...]

    def step(t, carry):
        h1c, h2c = carry
        gh1 = _vpu_mv(h1c, u0)
        gh2 = _vpu_mv(h2c, u1)
        h1n = _gates3(gi0_ref[pl.ds(t, 1), :], gh1[:, :_D],
                      gh1[:, _D:2 * _D], gh1[:, 2 * _D:] + bh0n, h1c)
        h2n = _gates3(gi1_ref[pl.ds(t, 1), :], gh2[:, :_D],
                      gh2[:, _D:2 * _D], gh2[:, 2 * _D:] + bh1n, h2c)
        ys0_ref[pl.ds(t, 1), :] = h1n
        f_ref[pl.ds(t, 1), :] = h2n
        return h1n, h2n

    h1c, h2c = lax.fori_loop(0, _CHUNK, step, (h1p, h2p), unroll=8)

    gi1_ref[...] = _dott(ys0_ref[...], w1t_ref[...]) + bi1_ref[...]

    h1_ref[...] = h1c
    h2_ref[...] = jnp.where(c == 0, jnp.zeros((1, _D), _F32), h2c)

    @pl.when(c == _NCHUNK)
    def _fin():
        hx_ref[...] = jnp.concatenate([h1p, h2c], axis=0)


def _nodes_scan(h, title, rnn_params):
    (wih0, whh0, bih0, bhh0), (wih1, whh1, bih1, bhh1) = rnn_params
    rz = jnp.concatenate([jnp.full((2 * _D,), 0.5, _F32),
                          jnp.ones((_D,), _F32)])
    w0t = wih0[:, :_D] * rz[:, None]
    w0h = wih0[:, _D:] * rz[:, None]
    w1s = wih1 * rz[:, None]
    u0 = whh0.T * rz[None, :]
    u1 = whh1.T * rz[None, :]
    zpad = jnp.zeros((_D,), _F32)
    bi0f = (bih0 + jnp.concatenate([bhh0[:2 * _D], zpad])) * rz
    bi1f = (bih1 + jnp.concatenate([bhh1[:2 * _D], zpad])) * rz
    bh0n = bhh0[2 * _D:].reshape(1, _D)
    bh1n = bhh1[2 * _D:].reshape(1, _D)

    full = lambda s: pl.BlockSpec(s, lambda c: (0,) * len(s))
    f, hx = pl.pallas_call(
        _nodes_body,
        grid=(_NCHUNK + 1,),
        in_specs=[
            pl.BlockSpec((_CHUNK, _D),
                         lambda c: (jnp.minimum(c, _NCHUNK - 1), 0)),
            full((1, _D)),
            full((_G3, _D)),
            full((_G3, _D)),
            full((_G3, _D)),
            full((1, _G3)),
            full((_D, _G3)),
            full((_D, _G3)),
            full((1, _D)),
            full((1, _G3)),
            full((1, _D)),
        ],
        out_specs=[
            pl.BlockSpec((_CHUNK, _D),
                         lambda c: (jnp.maximum(c - 1, 0), 0)),
            full((2, _D)),
        ],
        out_shape=[
            jax.ShapeDtypeStruct((_N, _D), _F32),
            jax.ShapeDtypeStruct((2, _D), _F32),
        ],
        scratch_shapes=[
            pltpu.VMEM((_CHUNK, _G3), _F32),
            pltpu.VMEM((_CHUNK, _G3), _F32),
            pltpu.VMEM((_CHUNK, _D), _F32),
            pltpu.VMEM((1, _D), _F32),
            pltpu.VMEM((1, _D), _F32),
            pltpu.VMEM((1, _G3), _F32),
        ],
        compiler_params=pltpu.CompilerParams(
            dimension_semantics=("arbitrary",),
        ),
    )(h, title, w0t, w0h, w1s, bi0f.reshape(1, -1), u0, u1,
      bh0n, bi1f.reshape(1, -1), bh1n)
    return f, hx


_NW = 16
_EW = _E // _NW
_NPAD = 10240
_SL = _NPAD // _NW
_VL = 16


def _edge_body(src_hbm, dst_hbm, sid_hbm, out_hbm,
               src_v, dst_v, cnt_v, tmp_v, acc_v, invs_v, invf_v, pred_v,
               sid_v, part_sh, inv_sh):
    cid = lax.axis_index("c")

    @pl.when(cid == 0)
    def _run():
        w = lax.axis_index("s")
        base = w * _EW
        pltpu.sync_copy(sid_hbm, sid_v)
        pltpu.sync_copy(src_hbm.at[pl.ds(base, _EW)], src_v)
        pltpu.sync_copy(dst_hbm.at[pl.ds(base, _EW)], dst_v)
        s0 = sid_v[...]

        zi = jnp.zeros((_VL,), jnp.int32)

        def zbody(i, carry):
            cnt_v[pl.ds(i * _VL, _VL)] = zi
            return carry

        lax.fori_loop(0, _NPAD // _VL, zbody, 0)

        ones = jnp.ones((_VL,), jnp.int32)

        def hbody(i, carry):
            s = src_v[pl.ds(i * _VL, _VL)]
            d = dst_v[pl.ds(i * _VL, _VL)]
            plsc.addupdate_scatter(cnt_v, [d], ones, mask=s == s0)
            return carry

        lax.fori_loop(0, _EW // _VL, hbody, 0)

        pltpu.sync_copy(cnt_v, part_sh.at[w])
        plsc.subcore_barrier()

        colbase = w * _SL
        pltpu.sync_copy(part_sh.at[0, pl.ds(colbase, _SL)], acc_v)

        def rbody(j, carry):
            pltpu.sync_copy(part_sh.at[j, pl.ds(colbase, _SL)], tmp_v)

            def abody(i, c2):
                sl = pl.ds(i * _VL, _VL)
                acc_v[sl] = acc_v[sl] + tmp_v[sl]
                return c2

            lax.fori_loop(0, _SL // _VL, abody, 0)
            return carry

        lax.fori_loop(1, _NW, rbody, 0)

        def ibody(i, carry):
            sl = pl.ds(i * _VL, _VL)
            invs_v[sl] = 1.0 / acc_v[sl].astype(_F32)
            return carry

        lax.fori_loop(0, _SL // _VL, ibody, 0)

        pltpu.sync_copy(invs_v, inv_sh.at[pl.ds(colbase, _SL)])
        plsc.subcore_barrier()
        pltpu.sync_copy(inv_sh, invf_v)

        zf = jnp.zeros((_VL,), _F32)

        def pbody(i, carry):
            sl = pl.ds(i * _VL, _VL)
            s = src_v[sl]
            d = dst_v[sl]
            vals = plsc.load_gather(invf_v, [d])
            pred_v[sl] = jnp.where(s == s0, vals, zf)
            return carry

        lax.fori_loop(0, _EW // _VL, pbody, 0)
        pltpu.sync_copy(pred_v, out_hbm.at[pl.ds(base, _EW)])


def _edge_softmax(src, dst, sid16):
    mesh = plsc.VectorSubcoreMesh(core_axis_name="c", subcore_axis_name="s")
    run = pl.kernel(
        _edge_body,
        out_type=jax.ShapeDtypeStruct((_E,), _F32),
        mesh=mesh,
        scratch_types=[
            pltpu.VMEM((_EW,), jnp.int32),
            pltpu.VMEM((_EW,), jnp.int32),
            pltpu.VMEM((_NPAD,), jnp.int32),
            pltpu.VMEM((_SL,), jnp.int32),
            pltpu.VMEM((_SL,), jnp.int32),
            pltpu.VMEM((_SL,), _F32),
            pltpu.VMEM((_NPAD,), _F32),
            pltpu.VMEM((_EW,), _F32),
            pltpu.VMEM((_VL,), jnp.int32),
            pltpu.VMEM_SHARED((_NW, _NPAD), jnp.int32),
            pltpu.VMEM_SHARED((_NPAD,), _F32),
        ],
        compiler_params=pltpu.CompilerParams(needs_layout_passes=False),
    )
    return run(src, dst, sid16)


def kernel(h, edge_index, title_src, src_node_id, glove, enc_params, rnn_params):
    emb = _gather_rows(glove, title_src)
    title = _title_encode(emb, enc_params)
    f, hx = _nodes_scan(h, title, rnn_params)
    sid16 = jnp.full((_VL,), src_node_id, jnp.int32)
    pred = _edge_softmax(edge_index[0], edge_index[1], sid16)
    return pred, hx, f


# speedup vs baseline: 1.2431x; 1.0369x over previous
import functools

import jax
import jax.numpy as jnp
from jax import lax
from jax.experimental import pallas as pl
from jax.experimental.pallas import tpu as pltpu
from jax.experimental.pallas import tpu_sc as plsc

_N = 10000
_E = 160000
_D = 128
_G3 = 384

_F32 = jnp.float32
_PREC = lax.Precision.HIGHEST


def _dott(a, b):
    return lax.dot_general(a, b, (((1,), (1,)), ((), ())),
                           precision=_PREC, preferred_element_type=_F32)


def _dot(a, b):
    return lax.dot_general(a, b, (((1,), (0,)), ((), ())),
                           precision=_PREC, preferred_element_type=_F32)


def _dott_fast(a, b):
    return lax.dot_general(a.astype(jnp.bfloat16), b.astype(jnp.bfloat16),
                           (((1,), (1,)), ((), ())),
                           precision=lax.Precision.DEFAULT,
                           preferred_element_type=_F32)


def _gates(gi, gh, hprev):
    r = jax.nn.sigmoid(gi[:, :_D] + gh[:, :_D])
    z = jax.nn.sigmoid(gi[:, _D:2 * _D] + gh[:, _D:2 * _D])
    n = jnp.tanh(gi[:, 2 * _D:] + r * gh[:, 2 * _D:])
    return (1.0 - z) * n + z * hprev


def _gather_body(ids_ref, glove_ref, out_ref):
    del ids_ref
    out_ref[...] = glove_ref[...]


def _gather_rows(glove, ids):
    L = ids.shape[0]
    glove3 = glove.reshape(glove.shape[0], 1, _D)
    grid_spec = pltpu.PrefetchScalarGridSpec(
        num_scalar_prefetch=1,
        grid=(L,),
        in_specs=[pl.BlockSpec((1, 1, _D), lambda i, ids: (ids[i], 0, 0))],
        out_specs=pl.BlockSpec((1, 1, _D), lambda i, ids: (i, 0, 0)),
    )
    out = pl.pallas_call(
        _gather_body,
        grid_spec=grid_spec,
        out_shape=jax.ShapeDtypeStruct((L, 1, _D), _F32),
    )(ids, glove3)
    return out.reshape(L, _D)


def _title_body(emb_ref,
                w0f_ref, u0f_ref, bi0f_ref, bh0f_ref,
                w0b_ref, u0b_ref, bi0b_ref, bh0b_ref,
                w1f_ref, u1f_ref, bi1f_ref, bh1f_ref,
                w1b_ref, u1b_ref, bi1b_ref, bh1b_ref,
                title_ref,
                ysf_ref, ysb_ref, g1f_ref, g1b_ref, g0f_ref, g0b_ref):
    L = emb_ref.shape[0]
    emb = emb_ref[...]
    g0f_ref[...] = _dott(emb, w0f_ref[...]) + bi0f_ref[...]
    g0b_ref[...] = _dott(emb, w0b_ref[...]) + bi0b_ref[...]
    u0f = u0f_ref[...]
    u0b = u0b_ref[...]
    bh0f = bh0f_ref[...]
    bh0b = bh0b_ref[...]

    def loop0(k, carry):
        hf, hb = carry
        t = L - 1 - k
        hf = _gates(g0f_ref[pl.ds(k, 1), :], _dot(hf, u0f) + bh0f, hf)
        hb = _gates(g0b_ref[pl.ds(t, 1), :], _dot(hb, u0b) + bh0b, hb)
        ysf_ref[pl.ds(k, 1), :] = hf
        ysb_ref[pl.ds(t, 1), :] = hb
        return hf, hb

    z = jnp.zeros((1, _D), _F32)
    h0f, h0b = lax.fori_loop(0, L, loop0, (z, z))

    l1 = jnp.concatenate([ysf_ref[...], ysb_ref[...]], axis=1)
    g1f_ref[...] = _dott(l1, w1f_ref[...]) + bi1f_ref[...]
    g1b_ref[...] = _dott(l1, w1b_ref[...]) + bi1b_ref[...]
    u1f = u1f_ref[...]
    u1b = u1b_ref[...]
    bh1f = bh1f_ref[...]
    bh1b = bh1b_ref[...]

    def loop1(k, carry):
        hf, hb = carry
        t = L - 1 - k
        hf = _gates(g1f_ref[pl.ds(k, 1), :], _dot(hf, u1f) + bh1f, hf)
        hb = _gates(g1b_ref[pl.ds(t, 1), :], _dot(hb, u1b) + bh1b, hb)
        return hf, hb

    h1f, h1b = lax.fori_loop(0, L, loop1, (z, z))
    title_ref[...] = 0.25 * (h0f + h0b + h1f + h1b)


def _title_encode(emb, enc_params):
    L = emb.shape[0]
    flat = []
    for (wih, whh, bih, bhh) in enc_params:
        flat += [wih, whh.T, bih.reshape(1, -1), bhh.reshape(1, -1)]
    full = lambda s: pl.BlockSpec(s, lambda: (0,) * len(s))
    in_specs = [full((L, _D))]
    for (wih, whh, bih, bhh) in enc_params:
        in_specs += [full(wih.shape), full(whh.T.shape),
                     full((1, _G3)), full((1, _G3))]
    return pl.pallas_call(
        _title_body,
        grid=(),
        in_specs=in_specs,
        out_specs=full((1, _D)),
        out_shape=jax.ShapeDtypeStruct((1, _D), _F32),
        scratch_shapes=[
            pltpu.VMEM((L, _D), _F32),
            pltpu.VMEM((L, _D), _F32),
            pltpu.VMEM((L, _G3), _F32),
            pltpu.VMEM((L, _G3), _F32),
            pltpu.VMEM((L, _G3), _F32),
            pltpu.VMEM((L, _G3), _F32),
        ],
    )(emb, *flat)


_CHUNK = 1000
_NCHUNK = _N // _CHUNK


def _gates3(gi, gr, gz, gn, hprev):
    r = 0.5 + 0.5 * jnp.tanh(gi[:, :_D] + gr)
    z = 0.5 + 0.5 * jnp.tanh(gi[:, _D:2 * _D] + gz)
    n = jnp.tanh(gi[:, 2 * _D:] + r * gn)
    return n + z * (hprev - n)


def _vpu_mv(hc, u):
    p = hc.reshape(_D, 1) * u
    p = p[0:64] + p[64:128]
    p = p[0:32] + p[32:64]
    p = p[0:16] + p[16:32]
    p = p[0:8] + p[8:16]
    return jnp.sum(p, axis=0, keepdims=True)


def _nodes_body(h_ref, title_ref, w0t_ref, w0h_ref, w1t_ref, bi0_ref,
                u0_ref, u1_ref, bh0n_ref, bi1_ref, bh1n_ref,
                f_ref, hx_ref,
                gi0_ref, gi1_ref, ys0_ref, h1_ref, h2_ref, gh2_ref, tp_ref):
    c = pl.program_id(0)

    @pl.when(c == 0)
    def _init():
        h1_ref[...] = jnp.zeros((1, _D), _F32)
        h2_ref[...] = jnp.zeros((1, _D), _F32)
        tp_ref[...] = _dott(title_ref[...], w0t_ref[...]) + bi0_ref[...]

    gi0_ref[...] = _dott_fast(h_ref[...], w0h_ref[...]) + tp_ref[...]

    u0 = u0_ref[...]
    u1 = u1_ref[...]
    bh0n = bh0n_ref[...]
    bh1n = bh1n_ref[...]

    h1p = h1_ref[...]
    h2p = h2_ref[...]
    gh2p0 = gh2_ref[...]

    def step(t, carry):
        h1c, h2c, gh2p = carry
        h2n = _gates3(gi1_ref[pl.ds(t, 1), :], gh2p[:, :_D],
                      gh2p[:, _D:2 * _D], gh2p[:, 2 * _D:] + bh1n, h2c)
        gh1 = _vpu_mv(h1c, u0)
        h1n = _gates3(gi0_ref[pl.ds(t, 1), :], gh1[:, :_D],
                      gh1[:, _D:2 * _D], gh1[:, 2 * _D:] + bh0n, h1c)
        gh2n = _vpu_mv(h2n, u1)
        ys0_ref[pl.ds(t, 1), :] = h1n
        f_ref[pl.ds(t, 1), :] = h2n
        return h1n, h2n, gh2n

    h1c, h2c, gh2c = lax.fori_loop(0, _CHUNK, step, (h1p, h2p, gh2p0),
                                   unroll=8)

    gi1_ref[...] = _dott_fast(ys0_ref[...], w1t_ref[...]) + bi1_ref[...]

    h1_ref[...] = h1c
    h2_ref[...] = jnp.where(c == 0, jnp.zeros((1, _D), _F32), h2c)
    gh2_ref[...] = jnp.where(c == 0, jnp.zeros((1, _G3), _F32), gh2c)

    @pl.when(c == _NCHUNK)
    def _fin():
        hx_ref[...] = jnp.concatenate([h1p, h2c], axis=0)


def _nodes_scan(h, title, rnn_params):
    (wih0, whh0, bih0, bhh0), (wih1, whh1, bih1, bhh1) = rnn_params
    rz = jnp.concatenate([jnp.full((2 * _D,), 0.5, _F32),
                          jnp.ones((_D,), _F32)])
    w0t = wih0[:, :_D] * rz[:, None]
    w0h = wih0[:, _D:] * rz[:, None]
    w1s = wih1 * rz[:, None]
    u0 = whh0.T * rz[None, :]
    u1 = whh1.T * rz[None, :]
    zpad = jnp.zeros((_D,), _F32)
    bi0f = (bih0 + jnp.concatenate([bhh0[:2 * _D], zpad])) * rz
    bi1f = (bih1 + jnp.concatenate([bhh1[:2 * _D], zpad])) * rz
    bh0n = bhh0[2 * _D:].reshape(1, _D)
    bh1n = bhh1[2 * _D:].reshape(1, _D)

    full = lambda s: pl.BlockSpec(s, lambda c: (0,) * len(s))
    f, hx = pl.pallas_call(
        _nodes_body,
        grid=(_NCHUNK + 1,),
        in_specs=[
            pl.BlockSpec((_CHUNK, _D),
                         lambda c: (jnp.minimum(c, _NCHUNK - 1), 0)),
            full((1, _D)),
            full((_G3, _D)),
            full((_G3, _D)),
            full((_G3, _D)),
            full((1, _G3)),
            full((_D, _G3)),
            full((_D, _G3)),
            full((1, _D)),
            full((1, _G3)),
            full((1, _D)),
        ],
        out_specs=[
            pl.BlockSpec((_CHUNK, _D),
                         lambda c: (jnp.maximum(c - 1, 0), 0)),
            full((2, _D)),
        ],
        out_shape=[
            jax.ShapeDtypeStruct((_N, _D), _F32),
            jax.ShapeDtypeStruct((2, _D), _F32),
        ],
        scratch_shapes=[
            pltpu.VMEM((_CHUNK, _G3), _F32),
            pltpu.VMEM((_CHUNK, _G3), _F32),
            pltpu.VMEM((_CHUNK, _D), _F32),
            pltpu.VMEM((1, _D), _F32),
            pltpu.VMEM((1, _D), _F32),
            pltpu.VMEM((1, _G3), _F32),
            pltpu.VMEM((1, _G3), _F32),
        ],
        compiler_params=pltpu.CompilerParams(
            dimension_semantics=("arbitrary",),
        ),
    )(h, title, w0t, w0h, w1s, bi0f.reshape(1, -1), u0, u1,
      bh0n, bi1f.reshape(1, -1), bh1n)
    return f, hx


_NW = 16
_EW = _E // _NW
_NPAD = 10240
_SL = _NPAD // _NW
_VL = 16


def _edge_body(src_hbm, dst_hbm, sid_hbm, out_hbm,
               src_v, dst_v, cnt_v, tmp_v, acc_v, invs_v, invf_v, pred_v,
               sid_v, part_sh, inv_sh):
    cid = lax.axis_index("c")

    @pl.when(cid == 0)
    def _run():
        w = lax.axis_index("s")
        base = w * _EW
        pltpu.sync_copy(sid_hbm, sid_v)
        pltpu.sync_copy(src_hbm.at[pl.ds(base, _EW)], src_v)
        pltpu.sync_copy(dst_hbm.at[pl.ds(base, _EW)], dst_v)
        s0 = sid_v[...]

        zi = jnp.zeros((_VL,), jnp.int32)

        def zbody(i, carry):
            cnt_v[pl.ds(i * _VL, _VL)] = zi
            return carry

        lax.fori_loop(0, _NPAD // _VL, zbody, 0)

        ones = jnp.ones((_VL,), jnp.int32)

        def hbody(i, carry):
            s = src_v[pl.ds(i * _VL, _VL)]
            d = dst_v[pl.ds(i * _VL, _VL)]
            plsc.addupdate_scatter(cnt_v, [d], ones, mask=s == s0)
            return carry

        lax.fori_loop(0, _EW // _VL, hbody, 0)

        pltpu.sync_copy(cnt_v, part_sh.at[w])
        plsc.subcore_barrier()

        colbase = w * _SL
        pltpu.sync_copy(part_sh.at[0, pl.ds(colbase, _SL)], acc_v)

        def rbody(j, carry):
            pltpu.sync_copy(part_sh.at[j, pl.ds(colbase, _SL)], tmp_v)

            def abody(i, c2):
                sl = pl.ds(i * _VL, _VL)
                acc_v[sl] = acc_v[sl] + tmp_v[sl]
                return c2

            lax.fori_loop(0, _SL // _VL, abody, 0)
            return carry

        lax.fori_loop(1, _NW, rbody, 0)

        def ibody(i, carry):
            sl = pl.ds(i * _VL, _VL)
            invs_v[sl] = 1.0 / acc_v[sl].astype(_F32)
            return carry

        lax.fori_loop(0, _SL // _VL, ibody, 0)

        pltpu.sync_copy(invs_v, inv_sh.at[pl.ds(colbase, _SL)])
        plsc.subcore_barrier()
        pltpu.sync_copy(inv_sh, invf_v)

        zf = jnp.zeros((_VL,), _F32)

        def pbody(i, carry):
            sl = pl.ds(i * _VL, _VL)
            s = src_v[sl]
            d = dst_v[sl]
            vals = plsc.load_gather(invf_v, [d])
            pred_v[sl] = jnp.where(s == s0, vals, zf)
            return carry

        lax.fori_loop(0, _EW // _VL, pbody, 0)
        pltpu.sync_copy(pred_v, out_hbm.at[pl.ds(base, _EW)])


def _edge_softmax(src, dst, sid16):
    mesh = plsc.VectorSubcoreMesh(core_axis_name="c", subcore_axis_name="s")
    run = pl.kernel(
        _edge_body,
        out_type=jax.ShapeDtypeStruct((_E,), _F32),
        mesh=mesh,
        scratch_types=[
            pltpu.VMEM((_EW,), jnp.int32),
            pltpu.VMEM((_EW,), jnp.int32),
            pltpu.VMEM((_NPAD,), jnp.int32),
            pltpu.VMEM((_SL,), jnp.int32),
            pltpu.VMEM((_SL,), jnp.int32),
            pltpu.VMEM((_SL,), _F32),
            pltpu.VMEM((_NPAD,), _F32),
            pltpu.VMEM((_EW,), _F32),
            pltpu.VMEM((_VL,), jnp.int32),
            pltpu.VMEM_SHARED((_NW, _NPAD), jnp.int32),
            pltpu.VMEM_SHARED((_NPAD,), _F32),
        ],
        compiler_params=pltpu.CompilerParams(needs_layout_passes=False),
    )
    return run(src, dst, sid16)


def kernel(h, edge_index, title_src, src_node_id, glove, enc_params, rnn_params):
    emb = _gather_rows(glove, title_src)
    title = _title_encode(emb, enc_params)
    f, hx = _nodes_scan(h, title, rnn_params)
    sid16 = jnp.full((_VL,), src_node_id, jnp.int32)
    pred = _edge_softmax(edge_index[0], edge_index[1], sid16)
    return pred, hx, f
